# baseline (speedup 1.0000x reference)
import functools

import jax
import jax.numpy as jnp
from jax import lax
from jax.experimental import pallas as pl
from jax.experimental.pallas import tpu as pltpu
from jax.experimental.pallas import tpu_sc as plsc

BATCH = 4096
SEQ = 200
WORD_D = 64
POS_D = 32
OUT_D = WORD_D + 2 * POS_D
N_TOK = BATCH * SEQ
N_POS = 200
P_ROWS = N_POS + 1

_INFO = plsc.get_sparse_core_info()
NC = _INFO.num_cores
NS = _INFO.num_subcores
NW = NC * NS
L = _INFO.num_lanes

TPW = N_TOK // NW
IDX_W = 128
CHUNK = 128
ROWS = CHUNK // IDX_W
N_CHUNK = TPW // CHUNK
ROWS_PW = TPW // IDX_W


def _embed_body(idx3, wtab, ptab, out,
                idx_v, fidx_v, wrows, prows, comb,
                isem0, isem1, gsem0, gsem1, wsem0, wsem1):
    c = lax.axis_index("c")
    s = lax.axis_index("s")
    wid = s * NC + c
    row_base = wid * ROWS_PW
    isem = (isem0, isem1)
    gsem = (gsem0, gsem1)
    wsem = (wsem0, wsem1)

    def idx_copy(g, b):
        return pltpu.make_async_copy(
            idx3.at[pl.ds(row_base + g * ROWS, ROWS)], idx_v.at[b], isem[b])

    def fuse_idx(b):
        for j in range(ROWS):
            for i in range(IDX_W // L):
                sl = pl.ds(i * L, L)
                p1 = idx_v[b, j, 1, sl]
                p2 = idx_v[b, j, 2, sl]
                fidx_v[b, j, sl] = p1 * P_ROWS + p2

    def gather_copies(b):
        cps = []
        for j in range(ROWS):
            r = pl.ds(j * IDX_W, IDX_W)
            cps.append(pltpu.make_async_copy(
                wtab.at[idx_v.at[b, j, 0]], wrows.at[b, r], gsem[b]))
            cps.append(pltpu.make_async_copy(
                ptab.at[fidx_v.at[b, j]], prows.at[b, r], gsem[b]))
        return cps

    def assemble(b):
        def body(t, carry):
            for k in range(WORD_D // L):
                comb[b, t, pl.ds(k * L, L)] = wrows[b, t, pl.ds(k * L, L)]
            for k in range(2 * POS_D // L):
                comb[b, t, pl.ds(WORD_D + k * L, L)] = \
                    prows[b, t, pl.ds(k * L, L)]
            return carry

        lax.fori_loop(0, CHUNK, body, 0, unroll=4)

    def write_copies(g, b):
        base = (row_base + g * ROWS) * IDX_W
        return [
            pltpu.make_async_copy(
                comb.at[b], out.at[pl.ds(base, CHUNK)], wsem[b]),
        ]

    idx_copy(0, 0).start()
    K = N_CHUNK // 2

    def step(k, carry):
        for b in range(2):
            g = 2 * k + b

            @pl.when(k >= 1)
            def _():
                for cp in write_copies(g - 2, b):
                    cp.wait()

            idx_copy(g, b).wait()
            fuse_idx(b)
            for cp in gather_copies(b):
                cp.start()

            bp = 1 - b
            if b == 0:
                @pl.when(k >= 1)
                def _():
                    for cp in gather_copies(bp):
                        cp.wait()
                    assemble(bp)
                    for cp in write_copies(2 * k - 1, bp):
                        cp.start()
                idx_copy(2 * k + 1, bp).start()
            else:
                for cp in gather_copies(bp):
                    cp.wait()
                assemble(bp)
                for cp in write_copies(2 * k, bp):
                    cp.start()

                @pl.when(k < K - 1)
                def _():
                    idx_copy(2 * k + 2, bp).start()
        return carry

    lax.fori_loop(0, K, step, 0)

    last = N_CHUNK - 1
    for cp in gather_copies(1):
        cp.wait()
    assemble(1)
    for cp in write_copies(last, 1):
        cp.start()
    for cp in write_copies(last - 1, 0):
        cp.wait()
    for cp in write_copies(last, 1):
        cp.wait()


@functools.partial(
    pl.kernel,
    out_type=jax.ShapeDtypeStruct((N_TOK, OUT_D), jnp.float32),
    mesh=plsc.VectorSubcoreMesh(core_axis_name="c", subcore_axis_name="s"),
    compiler_params=pltpu.CompilerParams(use_tc_tiling_on_sc=False),
    scratch_types=[
        pltpu.VMEM((2, ROWS, 3, IDX_W), jnp.int32),
        pltpu.VMEM((2, ROWS, IDX_W), jnp.int32),
        pltpu.VMEM((2, CHUNK, WORD_D), jnp.float32),
        pltpu.VMEM((2, CHUNK, 2 * POS_D), jnp.float32),
        pltpu.VMEM((2, CHUNK, OUT_D), jnp.float32),
        pltpu.SemaphoreType.DMA,
        pltpu.SemaphoreType.DMA,
        pltpu.SemaphoreType.DMA,
        pltpu.SemaphoreType.DMA,
        pltpu.SemaphoreType.DMA,
        pltpu.SemaphoreType.DMA,
    ],
)
def _embed(*args):
    _embed_body(*args)


def kernel(inputs, pos1, pos2, word_table, pos1_table, pos2_table):
    idx3 = jnp.stack([
        inputs.reshape(N_TOK // IDX_W, IDX_W),
        pos1.reshape(N_TOK // IDX_W, IDX_W),
        pos2.reshape(N_TOK // IDX_W, IDX_W),
    ], axis=1)
    ptab = jnp.concatenate([
        jnp.broadcast_to(pos1_table[:, None, :], (P_ROWS, P_ROWS, POS_D)),
        jnp.broadcast_to(pos2_table[None, :, :], (P_ROWS, P_ROWS, POS_D)),
    ], axis=-1).reshape(P_ROWS * P_ROWS, 2 * POS_D)
    out = _embed(idx3, word_table, ptab)
    return out.reshape(BATCH, SEQ, OUT_D)


# speedup vs baseline: 1.5406x; 1.5406x over previous
import functools

import jax
import jax.numpy as jnp
from jax import lax
from jax.experimental import pallas as pl
from jax.experimental.pallas import tpu as pltpu
from jax.experimental.pallas import tpu_sc as plsc

BATCH = 4096
SEQ = 200
WORD_D = 64
POS_D = 32
OUT_D = WORD_D + 2 * POS_D
N_TOK = BATCH * SEQ
N_POS = 200
P_ROWS = N_POS + 1

_INFO = plsc.get_sparse_core_info()
NC = _INFO.num_cores
NS = _INFO.num_subcores
NW = NC * NS
L = _INFO.num_lanes

TPW = N_TOK // NW
IDX_W = 128
CHUNK = 128
ROWS = CHUNK // IDX_W
N_CHUNK = TPW // CHUNK
ROWS_PW = TPW // IDX_W


def _embed_body(idx3, wtab, ptab, out,
                idx_v, fidx_v, wrows, prows, comb,
                isem0, isem1, gsem0, gsem1, wsem0, wsem1):
    c = lax.axis_index("c")
    s = lax.axis_index("s")
    wid = s * NC + c
    row_base = wid * ROWS_PW
    isem = (isem0, isem1)
    gsem = (gsem0, gsem1)
    wsem = (wsem0, wsem1)

    def idx_copy(g, b):
        return pltpu.make_async_copy(
            idx3.at[pl.ds(row_base + g * ROWS, ROWS)], idx_v.at[b], isem[b])

    def fuse_idx(b):
        for j in range(ROWS):
            for i in range(IDX_W // L):
                sl = pl.ds(i * L, L)
                p1 = idx_v[b, j, 1, sl]
                p2 = idx_v[b, j, 2, sl]
                fidx_v[b, j, sl] = p1 * P_ROWS + p2

    def gather_copies(b):
        cps = []
        for j in range(ROWS):
            r = pl.ds(j * IDX_W, IDX_W)
            cps.append(pltpu.make_async_copy(
                wtab.at[idx_v.at[b, j, 0]], wrows.at[b, r], gsem[b]))
            cps.append(pltpu.make_async_copy(
                ptab.at[fidx_v.at[b, j]], prows.at[b, r], gsem[b]))
        return cps

    def assemble(b):
        @plsc.parallel_loop(0, CHUNK, unroll=8)
        def body(t):
            for k in range(WORD_D // L):
                comb[b, t, pl.ds(k * L, L)] = wrows[b, t, pl.ds(k * L, L)]
            for k in range(2 * POS_D // L):
                comb[b, t, pl.ds(WORD_D + k * L, L)] = \
                    prows[b, t, pl.ds(k * L, L)]

    def write_copies(g, b):
        base = (row_base + g * ROWS) * IDX_W
        return [
            pltpu.make_async_copy(
                comb.at[b], out.at[pl.ds(base, CHUNK)], wsem[b]),
        ]

    idx_copy(0, 0).start()
    K = N_CHUNK // 2

    def step(k, carry):
        for b in range(2):
            g = 2 * k + b

            @pl.when(k >= 1)
            def _():
                for cp in write_copies(g - 2, b):
                    cp.wait()

            idx_copy(g, b).wait()
            fuse_idx(b)
            for cp in gather_copies(b):
                cp.start()

            bp = 1 - b
            if b == 0:
                @pl.when(k >= 1)
                def _():
                    for cp in gather_copies(bp):
                        cp.wait()
                    assemble(bp)
                    for cp in write_copies(2 * k - 1, bp):
                        cp.start()
                idx_copy(2 * k + 1, bp).start()
            else:
                for cp in gather_copies(bp):
                    cp.wait()
                assemble(bp)
                for cp in write_copies(2 * k, bp):
                    cp.start()

                @pl.when(k < K - 1)
                def _():
                    idx_copy(2 * k + 2, bp).start()
        return carry

    lax.fori_loop(0, K, step, 0)

    last = N_CHUNK - 1
    for cp in gather_copies(1):
        cp.wait()
    assemble(1)
    for cp in write_copies(last, 1):
        cp.start()
    for cp in write_copies(last - 1, 0):
        cp.wait()
    for cp in write_copies(last, 1):
        cp.wait()


@functools.partial(
    pl.kernel,
    out_type=jax.ShapeDtypeStruct((N_TOK, OUT_D), jnp.float32),
    mesh=plsc.VectorSubcoreMesh(core_axis_name="c", subcore_axis_name="s"),
    compiler_params=pltpu.CompilerParams(use_tc_tiling_on_sc=False),
    scratch_types=[
        pltpu.VMEM((2, ROWS, 3, IDX_W), jnp.int32),
        pltpu.VMEM((2, ROWS, IDX_W), jnp.int32),
        pltpu.VMEM((2, CHUNK, WORD_D), jnp.float32),
        pltpu.VMEM((2, CHUNK, 2 * POS_D), jnp.float32),
        pltpu.VMEM((2, CHUNK, OUT_D), jnp.float32),
        pltpu.SemaphoreType.DMA,
        pltpu.SemaphoreType.DMA,
        pltpu.SemaphoreType.DMA,
        pltpu.SemaphoreType.DMA,
        pltpu.SemaphoreType.DMA,
        pltpu.SemaphoreType.DMA,
    ],
)
def _embed(*args):
    _embed_body(*args)


def kernel(inputs, pos1, pos2, word_table, pos1_table, pos2_table):
    idx3 = jnp.stack([
        inputs.reshape(N_TOK // IDX_W, IDX_W),
        pos1.reshape(N_TOK // IDX_W, IDX_W),
        pos2.reshape(N_TOK // IDX_W, IDX_W),
    ], axis=1)
    ptab = jnp.concatenate([
        jnp.broadcast_to(pos1_table[:, None, :], (P_ROWS, P_ROWS, POS_D)),
        jnp.broadcast_to(pos2_table[None, :, :], (P_ROWS, P_ROWS, POS_D)),
    ], axis=-1).reshape(P_ROWS * P_ROWS, 2 * POS_D)
    out = _embed(idx3, word_table, ptab)
    return out.reshape(BATCH, SEQ, OUT_D)


# speedup vs baseline: 1.5954x; 1.0356x over previous
import functools

import jax
import jax.numpy as jnp
from jax import lax
from jax.experimental import pallas as pl
from jax.experimental.pallas import tpu as pltpu
from jax.experimental.pallas import tpu_sc as plsc

BATCH = 4096
SEQ = 200
WORD_D = 64
POS_D = 32
OUT_D = WORD_D + 2 * POS_D
N_TOK = BATCH * SEQ
N_POS = 200
P_ROWS = N_POS + 1

_INFO = plsc.get_sparse_core_info()
NC = _INFO.num_cores
NS = _INFO.num_subcores
NW = NC * NS
L = _INFO.num_lanes

TPW = N_TOK // NW
IDX_W = 128
CHUNK = 256
ROWS = CHUNK // IDX_W
N_CHUNK = TPW // CHUNK
ROWS_PW = TPW // IDX_W


def _embed_body(w_idx, p1_idx, p2_idx, wtab, ptab, out,
                widx_v, p1idx_v, p2idx_v, fidx_v, wrows, prows,
                isem0, isem1, gsem0, gsem1, wsem0, wsem1):
    c = lax.axis_index("c")
    s = lax.axis_index("s")
    wid = s * NC + c
    row_base = wid * ROWS_PW
    isem = (isem0, isem1)
    gsem = (gsem0, gsem1)
    wsem = (wsem0, wsem1)

    def idx_copies(g, b):
        r = pl.ds(row_base + g * ROWS, ROWS)
        return [
            pltpu.make_async_copy(w_idx.at[r], widx_v.at[b], isem[b]),
            pltpu.make_async_copy(p1_idx.at[r], p1idx_v.at[b], isem[b]),
            pltpu.make_async_copy(p2_idx.at[r], p2idx_v.at[b], isem[b]),
        ]

    def fuse_idx(b):
        for j in range(ROWS):
            for i in range(IDX_W // L):
                sl = pl.ds(i * L, L)
                p1 = p1idx_v[b, j, sl]
                p2 = p2idx_v[b, j, sl]
                fidx_v[b, j, sl] = p1 * P_ROWS + p2

    def gather_copies(b):
        cps = []
        for j in range(ROWS):
            r = pl.ds(j * IDX_W, IDX_W)
            cps.append(pltpu.make_async_copy(
                wtab.at[widx_v.at[b, j]], wrows.at[b, r], gsem[b]))
            cps.append(pltpu.make_async_copy(
                ptab.at[fidx_v.at[b, j]], prows.at[b, r], gsem[b]))
        return cps

    def write_copies(g, b):
        base = (row_base + g * ROWS) * IDX_W
        return [
            pltpu.make_async_copy(
                wrows.at[b],
                out.at[pl.ds(base, CHUNK), pl.ds(0, WORD_D)], wsem[b]),
            pltpu.make_async_copy(
                prows.at[b],
                out.at[pl.ds(base, CHUNK), pl.ds(WORD_D, 2 * POS_D)],
                wsem[b]),
        ]

    for cp in idx_copies(0, 0):
        cp.start()
    K = N_CHUNK // 2

    def step(k, carry):
        for b in range(2):
            g = 2 * k + b

            @pl.when(k >= 1)
            def _():
                for cp in write_copies(g - 2, b):
                    cp.wait()

            for cp in idx_copies(g, b):
                cp.wait()
            fuse_idx(b)
            for cp in gather_copies(b):
                cp.start()

            bp = 1 - b
            if b == 0:
                @pl.when(k >= 1)
                def _():
                    for cp in gather_copies(bp):
                        cp.wait()
                    for cp in write_copies(2 * k - 1, bp):
                        cp.start()
                for cp in idx_copies(2 * k + 1, bp):
                    cp.start()
            else:
                for cp in gather_copies(bp):
                    cp.wait()
                for cp in write_copies(2 * k, bp):
                    cp.start()

                @pl.when(k < K - 1)
                def _():
                    for cp in idx_copies(2 * k + 2, bp):
                        cp.start()
        return carry

    lax.fori_loop(0, K, step, 0)

    last = N_CHUNK - 1
    for cp in gather_copies(1):
        cp.wait()
    for cp in write_copies(last, 1):
        cp.start()
    for cp in write_copies(last - 1, 0):
        cp.wait()
    for cp in write_copies(last, 1):
        cp.wait()


@functools.partial(
    pl.kernel,
    out_type=jax.ShapeDtypeStruct((N_TOK, OUT_D), jnp.float32),
    mesh=plsc.VectorSubcoreMesh(core_axis_name="c", subcore_axis_name="s"),
    compiler_params=pltpu.CompilerParams(use_tc_tiling_on_sc=False),
    scratch_types=[
        pltpu.VMEM((2, ROWS, IDX_W), jnp.int32),
        pltpu.VMEM((2, ROWS, IDX_W), jnp.int32),
        pltpu.VMEM((2, ROWS, IDX_W), jnp.int32),
        pltpu.VMEM((2, ROWS, IDX_W), jnp.int32),
        pltpu.VMEM((2, CHUNK, WORD_D), jnp.float32),
        pltpu.VMEM((2, CHUNK, 2 * POS_D), jnp.float32),
        pltpu.SemaphoreType.DMA,
        pltpu.SemaphoreType.DMA,
        pltpu.SemaphoreType.DMA,
        pltpu.SemaphoreType.DMA,
        pltpu.SemaphoreType.DMA,
        pltpu.SemaphoreType.DMA,
    ],
)
def _embed(*args):
    _embed_body(*args)


def kernel(inputs, pos1, pos2, word_table, pos1_table, pos2_table):
    w_idx = inputs.reshape(N_TOK // IDX_W, IDX_W)
    p1_idx = pos1.reshape(N_TOK // IDX_W, IDX_W)
    p2_idx = pos2.reshape(N_TOK // IDX_W, IDX_W)
    ptab = jnp.concatenate([
        jnp.broadcast_to(pos1_table[:, None, :], (P_ROWS, P_ROWS, POS_D)),
        jnp.broadcast_to(pos2_table[None, :, :], (P_ROWS, P_ROWS, POS_D)),
    ], axis=-1).reshape(P_ROWS * P_ROWS, 2 * POS_D)
    out = _embed(w_idx, p1_idx, p2_idx, word_table, ptab)
    return out.reshape(BATCH, SEQ, OUT_D)
